# baseline (speedup 1.0000x reference)
import functools

import jax
import jax.numpy as jnp
from jax import lax
from jax.experimental import pallas as pl
from jax.experimental.pallas import tpu as pltpu
from jax.experimental.pallas import tpu_sc as plsc

NV = 4096
NG = 10000
NGP = 10240
NE = 160000
NEP = 163840
KEV = 128
WID = 64
NB = 4
SIG = 2.5

JB = 1024
IB = 512
NW = 32
EPW = NEP // NW
NCH = 40
CH = 128
RPT = NGP // 16
WSC = 64


def _rbf_body(v_ref, gp_ref, rbf_ref, rsum_ref, csum_ref):
    j = pl.program_id(0)
    v = v_ref[...]
    g = gp_ref[...]
    vn2 = jnp.sum(v * v, axis=1, keepdims=True)
    gn2 = jnp.sum(g * g, axis=1, keepdims=True)
    va = jnp.concatenate([-2.0 * v, jnp.ones((NV, 1), jnp.float32)], axis=1)
    ga = jnp.concatenate([g, gn2], axis=1)
    cross = lax.dot_general(va, ga, (((1,), (1,)), ((), ())),
                            preferred_element_type=jnp.float32)
    d2 = vn2 + cross
    dist = jnp.sqrt(jnp.maximum(d2, 0.0) + 1e-12)
    rbf = jnp.exp(dist * (-1.0 / SIG))
    rbf_ref[...] = rbf.astype(jnp.bfloat16)

    @pl.when(j == 0)
    def _():
        rsum_ref[...] = jnp.zeros_like(rsum_ref)

    rsum_ref[...] += jnp.sum(rbf, axis=1, keepdims=True)
    ones_col = jnp.ones((NV, 1), jnp.float32)
    csum_ref[...] = lax.dot_general(rbf, ones_col, (((0,), (0,)), ((), ())),
                                    preferred_element_type=jnp.float32)


def _diffblock_body(x_ref, ev_ref, mass_ref, evals_ref, t_ref,
                    w0_ref, b0_ref, w1_ref, b1_ref, out_ref):
    x = x_ref[...]
    ev = ev_ref[...]
    wx = ev * mass_ref[...]
    coeffs = lax.dot_general(wx, x, (((0,), (0,)), ((), ())),
                             preferred_element_type=jnp.float32)
    t = t_ref[...]
    sp = jnp.log(1.0 + jnp.exp(-jnp.abs(t))) + jnp.maximum(t, 0.0)
    decay = jnp.exp(-evals_ref[...] * sp)
    xd = jnp.dot(ev, coeffs * decay,
                 preferred_element_type=jnp.float32)
    h = jnp.maximum(
        jnp.dot(x, w0_ref[:WID, :], preferred_element_type=jnp.float32)
        + jnp.dot(xd, w0_ref[WID:, :], preferred_element_type=jnp.float32)
        + b0_ref[...], 0.0)
    out_ref[...] = x + jnp.dot(h, w1_ref[...],
                               preferred_element_type=jnp.float32) + b1_ref[...]


def _gin_body(rbf_ref, dx_ref, gx_ref, csum_ref, norm_ref, w_ref, y_ref):
    g = lax.dot_general(rbf_ref[...], dx_ref[...].astype(jnp.bfloat16),
                        (((0,), (0,)), ((), ())),
                        preferred_element_type=jnp.float32)
    g = g * (1.0 / (csum_ref[...] + 1e-6))
    g_in = gx_ref[...] + g
    y_ref[...] = jnp.dot(g_in, w_ref[...],
                         preferred_element_type=jnp.float32) * norm_ref[...]


def _h_body(zp_ref, y_ref, norm_ref, b_ref, w_ref, y2_ref):
    z = zp_ref[0] + zp_ref[1]
    norm = norm_ref[...]
    h = jnp.maximum(norm * (z + y_ref[...]) + b_ref[...], 0.0)
    y2_ref[...] = jnp.dot(h, w_ref[...],
                          preferred_element_type=jnp.float32) * norm


def _gx_body(zp_ref, y_ref, norm_ref, b_ref, gx_ref):
    z = zp_ref[0] + zp_ref[1]
    gx_ref[...] = norm_ref[...] * (z + y_ref[...]) + b_ref[...]


def _dupd_body(rbf_ref, gx_ref, rsum_ref, dx_ref, out_ref):
    acc = lax.dot_general(rbf_ref[...], gx_ref[...].astype(jnp.bfloat16),
                          (((1,), (0,)), ((), ())),
                          preferred_element_type=jnp.float32)
    out_ref[...] = dx_ref[...] + acc * (1.0 / (rsum_ref[...] + 1e-6))


def _init_body(sx_ref, w1_ref, b1_ref, gxin_ref, w2_ref, b2_ref,
               degp_ref, dx_ref, gx_ref, norm_ref):
    dx_ref[...] = jnp.dot(sx_ref[...], w1_ref[...],
                          preferred_element_type=jnp.float32) + b1_ref[...]
    gx_ref[...] = jnp.dot(gxin_ref[...], w2_ref[...],
                          preferred_element_type=jnp.float32) + b2_ref[...]
    deg = degp_ref[0, :, :1] + degp_ref[1, :, :1]
    norm_ref[...] = lax.rsqrt(deg + 1.0)


def _final_body(dx_ref, w_ref, b_ref, out_ref):
    out_ref[...] = jnp.dot(dx_ref[...], w_ref[...],
                           preferred_element_type=jnp.float32) + b_ref[...]


NBUF = 8


def _sc_scatter_body(y_hbm, src_hbm, dst_hbm, zeros_hbm, out_hbm,
                     sidx_v, didx_v, rows_v, z_sh,
                     s0, s1, s2, s3, s4, s5, s6, s7,
                     t0, t1, t2, t3, t4, t5, t6, t7):
    gsems = (s0, s1, s2, s3, s4, s5, s6, s7)
    ssems = (t0, t1, t2, t3, t4, t5, t6, t7)
    cid = lax.axis_index("c")
    sid = lax.axis_index("s")
    wid = sid * 2 + cid
    row0 = sid * RPT
    pltpu.sync_copy(zeros_hbm.at[pl.ds(row0, RPT)], z_sh.at[pl.ds(row0, RPT)])
    pltpu.sync_copy(src_hbm.at[wid], sidx_v)
    pltpu.sync_copy(dst_hbm.at[wid], didx_v)
    plsc.subcore_barrier()

    for b in range(NBUF):
        pltpu.async_copy(y_hbm.at[sidx_v.at[b]], rows_v.at[b], gsems[b])

    def outer(gi, carry):
        for b in range(NBUF):
            k = gi * NBUF + b
            pltpu.make_async_copy(y_hbm.at[sidx_v.at[k]], rows_v.at[b],
                                  gsems[b]).wait()
            pltpu.async_copy(rows_v.at[b], z_sh.at[didx_v.at[k]], ssems[b],
                             add=True)
            nxt = k + NBUF

            @pl.when(nxt < NCH)
            def _():
                pltpu.make_async_copy(rows_v.at[b], z_sh.at[didx_v.at[k]],
                                      ssems[b]).wait()
                pltpu.async_copy(y_hbm.at[sidx_v.at[nxt]], rows_v.at[b],
                                 gsems[b])
        return carry

    lax.fori_loop(0, NCH // NBUF, outer, 0)
    for b in range(NBUF):
        k = NCH - NBUF + b
        pltpu.make_async_copy(rows_v.at[b], z_sh.at[didx_v.at[k]],
                              ssems[b]).wait()
    plsc.subcore_barrier()
    pltpu.sync_copy(z_sh.at[pl.ds(row0, RPT)],
                    out_hbm.at[cid, pl.ds(row0, RPT)])


def _sc_scatter(y, src3, dst3, zeros2d):
    mesh = plsc.VectorSubcoreMesh(core_axis_name="c", subcore_axis_name="s")
    return pl.kernel(
        _sc_scatter_body,
        mesh=mesh,
        out_type=jax.ShapeDtypeStruct((2, NGP, WSC), jnp.float32),
        scratch_types=[
            pltpu.VMEM((NCH, CH), jnp.int32),
            pltpu.VMEM((NCH, CH), jnp.int32),
            pltpu.VMEM((NBUF, CH, WSC), jnp.float32),
            pltpu.VMEM_SHARED((NGP, WSC), jnp.float32),
        ] + [pltpu.SemaphoreType.DMA] * 16,
        compiler_params=pltpu.CompilerParams(use_tc_tiling_on_sc=False),
    )(y, src3, dst3, zeros2d)


def _rbf_pass(vertices, gp_pad):
    nj = NGP // JB
    return pl.pallas_call(
        _rbf_body,
        grid=(nj,),
        in_specs=[
            pl.BlockSpec((NV, 3), lambda j: (0, 0)),
            pl.BlockSpec((JB, 3), lambda j: (j, 0)),
        ],
        out_specs=[
            pl.BlockSpec((NV, JB), lambda j: (0, j)),
            pl.BlockSpec((NV, 1), lambda j: (0, 0)),
            pl.BlockSpec((JB, 1), lambda j: (j, 0)),
        ],
        out_shape=[
            jax.ShapeDtypeStruct((NV, NGP), jnp.bfloat16),
            jax.ShapeDtypeStruct((NV, 1), jnp.float32),
            jax.ShapeDtypeStruct((NGP, 1), jnp.float32),
        ],
    )(vertices, gp_pad)


def _diff_block(dx, evecs, mass_c, evals_c, t_r, W0, b0_r, W1, b1_r):
    return pl.pallas_call(
        _diffblock_body,
        out_shape=jax.ShapeDtypeStruct((NV, WID), jnp.float32),
    )(dx, evecs, mass_c, evals_c, t_r, W0, b0_r, W1, b1_r)


def _gin_pass(rbf, dx, gx, csum, norm, W):
    nj = NGP // JB
    return pl.pallas_call(
        _gin_body,
        grid=(nj,),
        in_specs=[
            pl.BlockSpec((NV, JB), lambda j: (0, j)),
            pl.BlockSpec((NV, WID), lambda j: (0, 0)),
            pl.BlockSpec((JB, WID), lambda j: (j, 0)),
            pl.BlockSpec((JB, 1), lambda j: (j, 0)),
            pl.BlockSpec((JB, 1), lambda j: (j, 0)),
            pl.BlockSpec((WID, WSC), lambda j: (0, 0)),
        ],
        out_specs=pl.BlockSpec((JB, WSC), lambda j: (j, 0)),
        out_shape=jax.ShapeDtypeStruct((NGP, WSC), jnp.float32),
    )(rbf, dx, gx, csum, norm, W)


def _h_pass(zp, y, norm, b_r, W):
    return pl.pallas_call(
        _h_body,
        out_shape=jax.ShapeDtypeStruct((NGP, WSC), jnp.float32),
    )(zp, y, norm, b_r, W)


def _gx_pass(zp, y, norm, b_r):
    return pl.pallas_call(
        _gx_body,
        out_shape=jax.ShapeDtypeStruct((NGP, WID), jnp.float32),
    )(zp, y, norm, b_r)


def _dupd_pass(rbf, gx, rsum, dx):
    ni = NV // IB
    return pl.pallas_call(
        _dupd_body,
        grid=(ni,),
        in_specs=[
            pl.BlockSpec((IB, NGP), lambda i: (i, 0)),
            pl.BlockSpec((NGP, WID), lambda i: (0, 0)),
            pl.BlockSpec((IB, 1), lambda i: (i, 0)),
            pl.BlockSpec((IB, WID), lambda i: (i, 0)),
        ],
        out_specs=pl.BlockSpec((IB, WID), lambda i: (i, 0)),
        out_shape=jax.ShapeDtypeStruct((NV, WID), jnp.float32),
    )(rbf, gx, rsum, dx)


def _init_pass(sx_pad, fl1W_pad, fl1b_r, gx_pad, fl2W, fl2b_r, degp):
    return pl.pallas_call(
        _init_body,
        out_shape=[
            jax.ShapeDtypeStruct((NV, WID), jnp.float32),
            jax.ShapeDtypeStruct((NGP, WID), jnp.float32),
            jax.ShapeDtypeStruct((NGP, 1), jnp.float32),
        ],
    )(sx_pad, fl1W_pad, fl1b_r, gx_pad, fl2W, fl2b_r, degp)


def _final_pass(dx, llW_pad, llb_pad):
    return pl.pallas_call(
        _final_body,
        out_shape=jax.ShapeDtypeStruct((NV, 128), jnp.float32),
    )(dx, llW_pad, llb_pad)


def kernel(surface_x, graph_x, vertices, graph_pos, edge_index, mass, evals,
           evecs, fl1_W, fl1_b, fl2_W, fl2_b, ll_W, ll_b, T,
           DW0, Db0, DW1, Db1, GW1, Gb1, GW2, Gb2):
    f32 = jnp.float32
    gp_pad = jnp.full((NGP, 3), 1e6, f32).at[:NG].set(graph_pos.astype(f32))
    gx_in_pad = jnp.zeros((NGP, 128), f32).at[:NG].set(graph_x.astype(f32))
    sx_pad = jnp.concatenate(
        [surface_x.astype(f32), jnp.zeros((NV, 3), f32)], axis=1)
    fl1W_pad = jnp.concatenate([fl1_W, jnp.zeros((3, WID), f32)], axis=0)
    ei = edge_index.astype(jnp.int32)
    pad_idx = jnp.full((NEP - NE,), NGP - 1, jnp.int32)
    src_p = jnp.concatenate([ei[0], pad_idx])
    dst_p = jnp.concatenate([ei[1], pad_idx])
    key = jnp.sort((src_p << 14) | dst_p)
    src_p = key >> 14
    dst_p = key & jnp.int32(16383)
    src3 = src_p.reshape(NW, NCH, CH)
    dst3 = dst_p.reshape(NW, NCH, CH)
    zeros2d = jnp.zeros((NGP, WSC), f32)
    mass_c = mass.reshape(NV, 1)
    evals_c = evals.reshape(KEV, 1)
    llW_pad = jnp.zeros((WID, 128), f32).at[:, :2].set(ll_W)
    llb_pad = jnp.zeros((1, 128), f32).at[:, :2].set(ll_b)
    b_row = lambda b: b.reshape(1, -1)

    degp = _sc_scatter(jnp.ones((NGP, WSC), f32), src3, dst3, zeros2d)

    rbf, rsum, csum = _rbf_pass(vertices.astype(f32), gp_pad)
    dx, gx, norm = _init_pass(sx_pad, fl1W_pad, b_row(fl1_b),
                              gx_in_pad, fl2_W, b_row(fl2_b), degp)

    for i in range(NB):
        dx = _diff_block(dx, evecs, mass_c, evals_c, b_row(T[i]),
                         DW0[i], b_row(Db0[i]), DW1[i], b_row(Db1[i]))
        y1 = _gin_pass(rbf, dx, gx, csum, norm, GW1[i])
        zp1 = _sc_scatter(y1, src3, dst3, zeros2d)
        y2 = _h_pass(zp1, y1, norm, b_row(Gb1[i]), GW2[i])
        zp2 = _sc_scatter(y2, src3, dst3, zeros2d)
        gx = _gx_pass(zp2, y2, norm, b_row(Gb2[i]))
        dx = _dupd_pass(rbf, gx, rsum, dx)

    out128 = _final_pass(dx, llW_pad, llb_pad)
    return out128[:, :2]


# speedup vs baseline: 1.0869x; 1.0869x over previous
import functools

import jax
import jax.numpy as jnp
from jax import lax
from jax.experimental import pallas as pl
from jax.experimental.pallas import tpu as pltpu
from jax.experimental.pallas import tpu_sc as plsc

NV = 4096
NG = 10000
NGP = 10240
NE = 160000
NEP = 163840
KEV = 128
WID = 64
NB = 4
SIG = 2.5

JB = 1024
IB = 512
NW = 32
EPW = NEP // NW
NCH = 40
CH = 128
RPT = NGP // 16
WSC = 64


def _rbf_body(v_ref, gp_ref, rbf_ref, rsum_ref, csum_ref):
    j = pl.program_id(0)
    v = v_ref[...]
    g = gp_ref[...]
    vn2 = jnp.sum(v * v, axis=1, keepdims=True)
    gn2 = jnp.sum(g * g, axis=1, keepdims=True)
    va = jnp.concatenate([-2.0 * v, jnp.ones((NV, 1), jnp.float32)], axis=1)
    ga = jnp.concatenate([g, gn2], axis=1)
    cross = lax.dot_general(va, ga, (((1,), (1,)), ((), ())),
                            preferred_element_type=jnp.float32)
    d2 = vn2 + cross
    dist = jnp.sqrt(jnp.maximum(d2, 0.0) + 1e-12)
    rbf = jnp.exp(dist * (-1.0 / SIG))
    rbf_ref[...] = rbf.astype(jnp.bfloat16)

    @pl.when(j == 0)
    def _():
        rsum_ref[...] = jnp.zeros_like(rsum_ref)

    rsum_ref[...] += jnp.sum(rbf, axis=1, keepdims=True)
    ones_col = jnp.ones((NV, 1), jnp.float32)
    csum_ref[...] = lax.dot_general(rbf, ones_col, (((0,), (0,)), ((), ())),
                                    preferred_element_type=jnp.float32)


def _diffblock_body(x_ref, ev_ref, mass_ref, evals_ref, t_ref,
                    w0_ref, b0_ref, w1_ref, b1_ref, out_ref):
    x = x_ref[...]
    ev = ev_ref[...]
    wx = ev * mass_ref[...]
    coeffs = lax.dot_general(wx, x, (((0,), (0,)), ((), ())),
                             preferred_element_type=jnp.float32)
    t = t_ref[...]
    sp = jnp.log(1.0 + jnp.exp(-jnp.abs(t))) + jnp.maximum(t, 0.0)
    decay = jnp.exp(-evals_ref[...] * sp)
    xd = jnp.dot(ev, coeffs * decay,
                 preferred_element_type=jnp.float32)
    h = jnp.maximum(
        jnp.dot(x, w0_ref[:WID, :], preferred_element_type=jnp.float32)
        + jnp.dot(xd, w0_ref[WID:, :], preferred_element_type=jnp.float32)
        + b0_ref[...], 0.0)
    out_ref[...] = x + jnp.dot(h, w1_ref[...],
                               preferred_element_type=jnp.float32) + b1_ref[...]


def _gin_body(rbf_ref, dx_ref, gx_ref, csum_ref, norm_ref, w_ref, y_ref):
    g = lax.dot_general(rbf_ref[...], dx_ref[...].astype(jnp.bfloat16),
                        (((0,), (0,)), ((), ())),
                        preferred_element_type=jnp.float32)
    g = g * (1.0 / (csum_ref[...] + 1e-6))
    g_in = gx_ref[...] + g
    y_ref[...] = jnp.dot(g_in, w_ref[...],
                         preferred_element_type=jnp.float32) * norm_ref[...]


def _h_body(zp_ref, y_ref, norm_ref, b_ref, w_ref, y2_ref):
    z = zp_ref[0] + zp_ref[1]
    norm = norm_ref[...]
    h = jnp.maximum(norm * (z + y_ref[...]) + b_ref[...], 0.0)
    y2_ref[...] = jnp.dot(h, w_ref[...],
                          preferred_element_type=jnp.float32) * norm


def _gx_body(zp_ref, y_ref, norm_ref, b_ref, gx_ref):
    z = zp_ref[0] + zp_ref[1]
    gx_ref[...] = norm_ref[...] * (z + y_ref[...]) + b_ref[...]


def _dupd_body(rbf_ref, gx_ref, rsum_ref, dx_ref, out_ref):
    acc = lax.dot_general(rbf_ref[...], gx_ref[...].astype(jnp.bfloat16),
                          (((1,), (0,)), ((), ())),
                          preferred_element_type=jnp.float32)
    out_ref[...] = dx_ref[...] + acc * (1.0 / (rsum_ref[...] + 1e-6))


def _init_body(sx_ref, w1_ref, b1_ref, gxin_ref, w2_ref, b2_ref,
               degp_ref, dx_ref, gx_ref, norm_ref):
    dx_ref[...] = jnp.dot(sx_ref[...], w1_ref[...],
                          preferred_element_type=jnp.float32) + b1_ref[...]
    gx_ref[...] = jnp.dot(gxin_ref[...], w2_ref[...],
                          preferred_element_type=jnp.float32) + b2_ref[...]
    deg = degp_ref[0, :, :1] + degp_ref[1, :, :1]
    norm_ref[...] = lax.rsqrt(deg + 1.0)


def _final_body(dx_ref, w_ref, b_ref, out_ref):
    out_ref[...] = jnp.dot(dx_ref[...], w_ref[...],
                           preferred_element_type=jnp.float32) + b_ref[...]


NBUF = 8


def _sc_scatter_body(y_hbm, src_hbm, dst_hbm, zeros_hbm, out_hbm,
                     sidx_v, didx_v, rows_v, z_sh,
                     s0, s1, s2, s3, s4, s5, s6, s7,
                     t0, t1, t2, t3, t4, t5, t6, t7):
    gsems = (s0, s1, s2, s3, s4, s5, s6, s7)
    ssems = (t0, t1, t2, t3, t4, t5, t6, t7)
    cid = lax.axis_index("c")
    sid = lax.axis_index("s")
    wid = sid * 2 + cid
    row0 = sid * RPT
    pltpu.sync_copy(zeros_hbm.at[pl.ds(row0, RPT)], z_sh.at[pl.ds(row0, RPT)])
    pltpu.sync_copy(src_hbm.at[wid], sidx_v)
    pltpu.sync_copy(dst_hbm.at[wid], didx_v)
    plsc.subcore_barrier()

    for b in range(NBUF):
        pltpu.async_copy(y_hbm.at[sidx_v.at[b]], rows_v.at[b], gsems[b])

    def outer(gi, carry):
        for b in range(NBUF):
            k = gi * NBUF + b
            pltpu.make_async_copy(y_hbm.at[sidx_v.at[k]], rows_v.at[b],
                                  gsems[b]).wait()
            pltpu.async_copy(rows_v.at[b], z_sh.at[didx_v.at[k]], ssems[b],
                             add=True)
            nxt = k + NBUF

            @pl.when(nxt < NCH)
            def _():
                pltpu.make_async_copy(rows_v.at[b], z_sh.at[didx_v.at[k]],
                                      ssems[b]).wait()
                pltpu.async_copy(y_hbm.at[sidx_v.at[nxt]], rows_v.at[b],
                                 gsems[b])
        return carry

    lax.fori_loop(0, NCH // NBUF, outer, 0)
    for b in range(NBUF):
        k = NCH - NBUF + b
        pltpu.make_async_copy(rows_v.at[b], z_sh.at[didx_v.at[k]],
                              ssems[b]).wait()
    plsc.subcore_barrier()
    pltpu.sync_copy(z_sh.at[pl.ds(row0, RPT)],
                    out_hbm.at[cid, pl.ds(row0, RPT)])


def _sc_scatter(y, src3, dst3, zeros2d):
    mesh = plsc.VectorSubcoreMesh(core_axis_name="c", subcore_axis_name="s")
    return pl.kernel(
        _sc_scatter_body,
        mesh=mesh,
        out_type=jax.ShapeDtypeStruct((2, NGP, WSC), jnp.float32),
        scratch_types=[
            pltpu.VMEM((NCH, CH), jnp.int32),
            pltpu.VMEM((NCH, CH), jnp.int32),
            pltpu.VMEM((NBUF, CH, WSC), jnp.float32),
            pltpu.VMEM_SHARED((NGP, WSC), jnp.float32),
        ] + [pltpu.SemaphoreType.DMA] * 16,
        compiler_params=pltpu.CompilerParams(use_tc_tiling_on_sc=False),
    )(y, src3, dst3, zeros2d)


def _rbf_pass(vertices, gp_pad):
    nj = NGP // JB
    return pl.pallas_call(
        _rbf_body,
        grid=(nj,),
        in_specs=[
            pl.BlockSpec((NV, 3), lambda j: (0, 0)),
            pl.BlockSpec((JB, 3), lambda j: (j, 0)),
        ],
        out_specs=[
            pl.BlockSpec((NV, JB), lambda j: (0, j)),
            pl.BlockSpec((NV, 1), lambda j: (0, 0)),
            pl.BlockSpec((JB, 1), lambda j: (j, 0)),
        ],
        out_shape=[
            jax.ShapeDtypeStruct((NV, NGP), jnp.bfloat16),
            jax.ShapeDtypeStruct((NV, 1), jnp.float32),
            jax.ShapeDtypeStruct((NGP, 1), jnp.float32),
        ],
    )(vertices, gp_pad)


def _diff_block(dx, evecs, mass_c, evals_c, t_r, W0, b0_r, W1, b1_r):
    return pl.pallas_call(
        _diffblock_body,
        out_shape=jax.ShapeDtypeStruct((NV, WID), jnp.float32),
    )(dx, evecs, mass_c, evals_c, t_r, W0, b0_r, W1, b1_r)


def _gin_pass(rbf, dx, gx, csum, norm, W):
    nj = NGP // JB
    return pl.pallas_call(
        _gin_body,
        grid=(nj,),
        in_specs=[
            pl.BlockSpec((NV, JB), lambda j: (0, j)),
            pl.BlockSpec((NV, WID), lambda j: (0, 0)),
            pl.BlockSpec((JB, WID), lambda j: (j, 0)),
            pl.BlockSpec((JB, 1), lambda j: (j, 0)),
            pl.BlockSpec((JB, 1), lambda j: (j, 0)),
            pl.BlockSpec((WID, WSC), lambda j: (0, 0)),
        ],
        out_specs=pl.BlockSpec((JB, WSC), lambda j: (j, 0)),
        out_shape=jax.ShapeDtypeStruct((NGP, WSC), jnp.float32),
    )(rbf, dx, gx, csum, norm, W)


def _h_pass(zp, y, norm, b_r, W):
    return pl.pallas_call(
        _h_body,
        out_shape=jax.ShapeDtypeStruct((NGP, WSC), jnp.float32),
    )(zp, y, norm, b_r, W)


def _gx_pass(zp, y, norm, b_r):
    return pl.pallas_call(
        _gx_body,
        out_shape=jax.ShapeDtypeStruct((NGP, WID), jnp.float32),
    )(zp, y, norm, b_r)


def _dupd_pass(rbf, gx, rsum, dx):
    ni = NV // IB
    return pl.pallas_call(
        _dupd_body,
        grid=(ni,),
        in_specs=[
            pl.BlockSpec((IB, NGP), lambda i: (i, 0)),
            pl.BlockSpec((NGP, WID), lambda i: (0, 0)),
            pl.BlockSpec((IB, 1), lambda i: (i, 0)),
            pl.BlockSpec((IB, WID), lambda i: (i, 0)),
        ],
        out_specs=pl.BlockSpec((IB, WID), lambda i: (i, 0)),
        out_shape=jax.ShapeDtypeStruct((NV, WID), jnp.float32),
    )(rbf, gx, rsum, dx)


def _init_pass(sx_pad, fl1W_pad, fl1b_r, gx_pad, fl2W, fl2b_r, degp):
    return pl.pallas_call(
        _init_body,
        out_shape=[
            jax.ShapeDtypeStruct((NV, WID), jnp.float32),
            jax.ShapeDtypeStruct((NGP, WID), jnp.float32),
            jax.ShapeDtypeStruct((NGP, 1), jnp.float32),
        ],
    )(sx_pad, fl1W_pad, fl1b_r, gx_pad, fl2W, fl2b_r, degp)


def _final_pass(dx, llW_pad, llb_pad):
    return pl.pallas_call(
        _final_body,
        out_shape=jax.ShapeDtypeStruct((NV, 128), jnp.float32),
    )(dx, llW_pad, llb_pad)


def kernel(surface_x, graph_x, vertices, graph_pos, edge_index, mass, evals,
           evecs, fl1_W, fl1_b, fl2_W, fl2_b, ll_W, ll_b, T,
           DW0, Db0, DW1, Db1, GW1, Gb1, GW2, Gb2):
    f32 = jnp.float32
    gp_pad = jnp.full((NGP, 3), 1e6, f32).at[:NG].set(graph_pos.astype(f32))
    gx_in_pad = jnp.zeros((NGP, 128), f32).at[:NG].set(graph_x.astype(f32))
    sx_pad = jnp.concatenate(
        [surface_x.astype(f32), jnp.zeros((NV, 3), f32)], axis=1)
    fl1W_pad = jnp.concatenate([fl1_W, jnp.zeros((3, WID), f32)], axis=0)
    ei = edge_index.astype(jnp.int32)
    pad_idx = jnp.full((NEP - NE,), NGP - 1, jnp.int32)
    src3 = jnp.concatenate([ei[0], pad_idx]).reshape(NW, NCH, CH)
    dst3 = jnp.concatenate([ei[1], pad_idx]).reshape(NW, NCH, CH)
    zeros2d = jnp.zeros((NGP, WSC), f32)
    mass_c = mass.reshape(NV, 1)
    evals_c = evals.reshape(KEV, 1)
    llW_pad = jnp.zeros((WID, 128), f32).at[:, :2].set(ll_W)
    llb_pad = jnp.zeros((1, 128), f32).at[:, :2].set(ll_b)
    b_row = lambda b: b.reshape(1, -1)

    degp = _sc_scatter(jnp.ones((NGP, WSC), f32), src3, dst3, zeros2d)

    rbf, rsum, csum = _rbf_pass(vertices.astype(f32), gp_pad)
    dx, gx, norm = _init_pass(sx_pad, fl1W_pad, b_row(fl1_b),
                              gx_in_pad, fl2_W, b_row(fl2_b), degp)

    for i in range(NB):
        dx = _diff_block(dx, evecs, mass_c, evals_c, b_row(T[i]),
                         DW0[i], b_row(Db0[i]), DW1[i], b_row(Db1[i]))
        y1 = _gin_pass(rbf, dx, gx, csum, norm, GW1[i])
        zp1 = _sc_scatter(y1, src3, dst3, zeros2d)
        y2 = _h_pass(zp1, y1, norm, b_row(Gb1[i]), GW2[i])
        zp2 = _sc_scatter(y2, src3, dst3, zeros2d)
        gx = _gx_pass(zp2, y2, norm, b_row(Gb2[i]))
        dx = _dupd_pass(rbf, gx, rsum, dx)

    out128 = _final_pass(dx, llW_pad, llb_pad)
    return out128[:, :2]


# speedup vs baseline: 1.1020x; 1.0139x over previous
import functools

import jax
import jax.numpy as jnp
from jax import lax
from jax.experimental import pallas as pl
from jax.experimental.pallas import tpu as pltpu
from jax.experimental.pallas import tpu_sc as plsc

NV = 4096
NG = 10000
NGP = 10240
NE = 160000
NEP = 163840
KEV = 128
WID = 64
NB = 4
SIG = 2.5

JB = 1024
IB = 512
NW = 32
EPW = NEP // NW
NCH = 40
CH = 128
RPT = NGP // 16
WSC = 64


def _rbf_body(v_ref, gp_ref, rbf_ref, rsum_ref, csum_ref):
    j = pl.program_id(0)
    v = v_ref[...]
    g = gp_ref[...]
    vn2 = jnp.sum(v * v, axis=1, keepdims=True)
    gn2 = jnp.sum(g * g, axis=1, keepdims=True)
    va = jnp.concatenate([-2.0 * v, jnp.ones((NV, 1), jnp.float32)], axis=1)
    ga = jnp.concatenate([g, gn2], axis=1)
    cross = lax.dot_general(va, ga, (((1,), (1,)), ((), ())),
                            preferred_element_type=jnp.float32)
    d2 = vn2 + cross
    dist = jnp.sqrt(jnp.maximum(d2, 0.0) + 1e-12)
    rbf = jnp.exp(dist * (-1.0 / SIG))
    rbf_ref[...] = rbf.astype(jnp.bfloat16)

    @pl.when(j == 0)
    def _():
        rsum_ref[...] = jnp.zeros_like(rsum_ref)

    rsum_ref[...] += jnp.sum(rbf, axis=1, keepdims=True)
    ones_col = jnp.ones((NV, 1), jnp.float32)
    csum_ref[...] = lax.dot_general(rbf, ones_col, (((0,), (0,)), ((), ())),
                                    preferred_element_type=jnp.float32)


def _diffblock_body(x_ref, ev_ref, mass_ref, evals_ref, t_ref,
                    w0_ref, b0_ref, w1_ref, b1_ref, out_ref):
    x = x_ref[...]
    ev = ev_ref[...]
    wx = ev * mass_ref[...]
    coeffs = lax.dot_general(wx, x, (((0,), (0,)), ((), ())),
                             preferred_element_type=jnp.float32)
    t = t_ref[...]
    sp = jnp.log(1.0 + jnp.exp(-jnp.abs(t))) + jnp.maximum(t, 0.0)
    decay = jnp.exp(-evals_ref[...] * sp)
    xd = jnp.dot(ev, coeffs * decay,
                 preferred_element_type=jnp.float32)
    h = jnp.maximum(
        jnp.dot(x, w0_ref[:WID, :], preferred_element_type=jnp.float32)
        + jnp.dot(xd, w0_ref[WID:, :], preferred_element_type=jnp.float32)
        + b0_ref[...], 0.0)
    out_ref[...] = x + jnp.dot(h, w1_ref[...],
                               preferred_element_type=jnp.float32) + b1_ref[...]


def _gin_body(rbf_ref, dx_ref, gx_ref, csum_ref, norm_ref, w_ref, y_ref):
    g = lax.dot_general(rbf_ref[...], dx_ref[...].astype(jnp.bfloat16),
                        (((0,), (0,)), ((), ())),
                        preferred_element_type=jnp.float32)
    g = g * (1.0 / (csum_ref[...] + 1e-6))
    g_in = gx_ref[...] + g
    y_ref[...] = jnp.dot(g_in, w_ref[...],
                         preferred_element_type=jnp.float32) * norm_ref[...]


def _h_body(zp_ref, y_ref, norm_ref, b_ref, w_ref, y2_ref):
    z = zp_ref[0] + zp_ref[1]
    norm = norm_ref[...]
    h = jnp.maximum(norm * (z + y_ref[...]) + b_ref[...], 0.0)
    y2_ref[...] = jnp.dot(h, w_ref[...],
                          preferred_element_type=jnp.float32) * norm


def _dupd_body(rbf_ref, zp_ref, y_ref, norm_ref, b_ref, rsum_ref, dx_ref,
               out_ref, gx_out, gx_sc):
    i = pl.program_id(0)

    @pl.when(i == 0)
    def _():
        z = zp_ref[0] + zp_ref[1]
        gxv = norm_ref[...] * (z + y_ref[...]) + b_ref[...]
        gx_out[...] = gxv
        gx_sc[...] = gxv.astype(jnp.bfloat16)

    acc = lax.dot_general(rbf_ref[...], gx_sc[...], (((1,), (0,)), ((), ())),
                          preferred_element_type=jnp.float32)
    out_ref[...] = dx_ref[...] + acc * (1.0 / (rsum_ref[...] + 1e-6))


def _init_body(sx_ref, w1_ref, b1_ref, gxin_ref, w2_ref, b2_ref,
               degp_ref, dx_ref, gx_ref, norm_ref):
    dx_ref[...] = jnp.dot(sx_ref[...], w1_ref[...],
                          preferred_element_type=jnp.float32) + b1_ref[...]
    gx_ref[...] = jnp.dot(gxin_ref[...], w2_ref[...],
                          preferred_element_type=jnp.float32) + b2_ref[...]
    deg = degp_ref[0, :, :1] + degp_ref[1, :, :1]
    norm_ref[...] = lax.rsqrt(deg + 1.0)


def _final_body(dx_ref, w_ref, b_ref, out_ref):
    out_ref[...] = jnp.dot(dx_ref[...], w_ref[...],
                           preferred_element_type=jnp.float32) + b_ref[...]


NBUF = 8


def _sc_scatter_body(y_hbm, src_hbm, dst_hbm, zeros_hbm, out_hbm,
                     sidx_v, didx_v, rows_v, z_sh,
                     s0, s1, s2, s3, s4, s5, s6, s7,
                     t0, t1, t2, t3, t4, t5, t6, t7):
    gsems = (s0, s1, s2, s3, s4, s5, s6, s7)
    ssems = (t0, t1, t2, t3, t4, t5, t6, t7)
    cid = lax.axis_index("c")
    sid = lax.axis_index("s")
    wid = sid * 2 + cid
    row0 = sid * RPT
    pltpu.sync_copy(zeros_hbm.at[pl.ds(row0, RPT)], z_sh.at[pl.ds(row0, RPT)])
    pltpu.sync_copy(src_hbm.at[wid], sidx_v)
    pltpu.sync_copy(dst_hbm.at[wid], didx_v)
    plsc.subcore_barrier()

    for b in range(NBUF):
        pltpu.async_copy(y_hbm.at[sidx_v.at[b]], rows_v.at[b], gsems[b])

    def outer(gi, carry):
        for b in range(NBUF):
            k = gi * NBUF + b
            pltpu.make_async_copy(y_hbm.at[sidx_v.at[k]], rows_v.at[b],
                                  gsems[b]).wait()
            pltpu.async_copy(rows_v.at[b], z_sh.at[didx_v.at[k]], ssems[b],
                             add=True)
            nxt = k + NBUF

            @pl.when(nxt < NCH)
            def _():
                pltpu.make_async_copy(rows_v.at[b], z_sh.at[didx_v.at[k]],
                                      ssems[b]).wait()
                pltpu.async_copy(y_hbm.at[sidx_v.at[nxt]], rows_v.at[b],
                                 gsems[b])
        return carry

    lax.fori_loop(0, NCH // NBUF, outer, 0)
    for b in range(NBUF):
        k = NCH - NBUF + b
        pltpu.make_async_copy(rows_v.at[b], z_sh.at[didx_v.at[k]],
                              ssems[b]).wait()
    plsc.subcore_barrier()
    pltpu.sync_copy(z_sh.at[pl.ds(row0, RPT)],
                    out_hbm.at[cid, pl.ds(row0, RPT)])


def _sc_scatter(y, src3, dst3, zeros2d):
    mesh = plsc.VectorSubcoreMesh(core_axis_name="c", subcore_axis_name="s")
    return pl.kernel(
        _sc_scatter_body,
        mesh=mesh,
        out_type=jax.ShapeDtypeStruct((2, NGP, WSC), jnp.float32),
        scratch_types=[
            pltpu.VMEM((NCH, CH), jnp.int32),
            pltpu.VMEM((NCH, CH), jnp.int32),
            pltpu.VMEM((NBUF, CH, WSC), jnp.float32),
            pltpu.VMEM_SHARED((NGP, WSC), jnp.float32),
        ] + [pltpu.SemaphoreType.DMA] * 16,
        compiler_params=pltpu.CompilerParams(use_tc_tiling_on_sc=False),
    )(y, src3, dst3, zeros2d)


def _rbf_pass(vertices, gp_pad):
    nj = NGP // JB
    return pl.pallas_call(
        _rbf_body,
        grid=(nj,),
        in_specs=[
            pl.BlockSpec((NV, 3), lambda j: (0, 0)),
            pl.BlockSpec((JB, 3), lambda j: (j, 0)),
        ],
        out_specs=[
            pl.BlockSpec((NV, JB), lambda j: (0, j)),
            pl.BlockSpec((NV, 1), lambda j: (0, 0)),
            pl.BlockSpec((JB, 1), lambda j: (j, 0)),
        ],
        out_shape=[
            jax.ShapeDtypeStruct((NV, NGP), jnp.bfloat16),
            jax.ShapeDtypeStruct((NV, 1), jnp.float32),
            jax.ShapeDtypeStruct((NGP, 1), jnp.float32),
        ],
    )(vertices, gp_pad)


def _diff_block(dx, evecs, mass_c, evals_c, t_r, W0, b0_r, W1, b1_r):
    return pl.pallas_call(
        _diffblock_body,
        out_shape=jax.ShapeDtypeStruct((NV, WID), jnp.float32),
    )(dx, evecs, mass_c, evals_c, t_r, W0, b0_r, W1, b1_r)


def _gin_pass(rbf, dx, gx, csum, norm, W):
    nj = NGP // JB
    return pl.pallas_call(
        _gin_body,
        grid=(nj,),
        in_specs=[
            pl.BlockSpec((NV, JB), lambda j: (0, j)),
            pl.BlockSpec((NV, WID), lambda j: (0, 0)),
            pl.BlockSpec((JB, WID), lambda j: (j, 0)),
            pl.BlockSpec((JB, 1), lambda j: (j, 0)),
            pl.BlockSpec((JB, 1), lambda j: (j, 0)),
            pl.BlockSpec((WID, WSC), lambda j: (0, 0)),
        ],
        out_specs=pl.BlockSpec((JB, WSC), lambda j: (j, 0)),
        out_shape=jax.ShapeDtypeStruct((NGP, WSC), jnp.float32),
    )(rbf, dx, gx, csum, norm, W)


def _h_pass(zp, y, norm, b_r, W):
    return pl.pallas_call(
        _h_body,
        out_shape=jax.ShapeDtypeStruct((NGP, WSC), jnp.float32),
    )(zp, y, norm, b_r, W)


def _dupd_pass(rbf, zp, y2, norm, b_r, rsum, dx):
    ni = NV // IB
    return pl.pallas_call(
        _dupd_body,
        grid=(ni,),
        in_specs=[
            pl.BlockSpec((IB, NGP), lambda i: (i, 0)),
            pl.BlockSpec((2, NGP, WID), lambda i: (0, 0, 0)),
            pl.BlockSpec((NGP, WID), lambda i: (0, 0)),
            pl.BlockSpec((NGP, 1), lambda i: (0, 0)),
            pl.BlockSpec((1, WID), lambda i: (0, 0)),
            pl.BlockSpec((IB, 1), lambda i: (i, 0)),
            pl.BlockSpec((IB, WID), lambda i: (i, 0)),
        ],
        out_specs=[
            pl.BlockSpec((IB, WID), lambda i: (i, 0)),
            pl.BlockSpec((NGP, WID), lambda i: (0, 0)),
        ],
        out_shape=[
            jax.ShapeDtypeStruct((NV, WID), jnp.float32),
            jax.ShapeDtypeStruct((NGP, WID), jnp.float32),
        ],
        scratch_shapes=[pltpu.VMEM((NGP, WID), jnp.bfloat16)],
    )(rbf, zp, y2, norm, b_r, rsum, dx)


def _init_pass(sx_pad, fl1W_pad, fl1b_r, gx_pad, fl2W, fl2b_r, degp):
    return pl.pallas_call(
        _init_body,
        out_shape=[
            jax.ShapeDtypeStruct((NV, WID), jnp.float32),
            jax.ShapeDtypeStruct((NGP, WID), jnp.float32),
            jax.ShapeDtypeStruct((NGP, 1), jnp.float32),
        ],
    )(sx_pad, fl1W_pad, fl1b_r, gx_pad, fl2W, fl2b_r, degp)


def _final_pass(dx, llW_pad, llb_pad):
    return pl.pallas_call(
        _final_body,
        out_shape=jax.ShapeDtypeStruct((NV, 128), jnp.float32),
    )(dx, llW_pad, llb_pad)


def kernel(surface_x, graph_x, vertices, graph_pos, edge_index, mass, evals,
           evecs, fl1_W, fl1_b, fl2_W, fl2_b, ll_W, ll_b, T,
           DW0, Db0, DW1, Db1, GW1, Gb1, GW2, Gb2):
    f32 = jnp.float32
    gp_pad = jnp.full((NGP, 3), 1e6, f32).at[:NG].set(graph_pos.astype(f32))
    gx_in_pad = jnp.zeros((NGP, 128), f32).at[:NG].set(graph_x.astype(f32))
    sx_pad = jnp.concatenate(
        [surface_x.astype(f32), jnp.zeros((NV, 3), f32)], axis=1)
    fl1W_pad = jnp.concatenate([fl1_W, jnp.zeros((3, WID), f32)], axis=0)
    ei = edge_index.astype(jnp.int32)
    pad_idx = jnp.full((NEP - NE,), NGP - 1, jnp.int32)
    src3 = jnp.concatenate([ei[0], pad_idx]).reshape(NW, NCH, CH)
    dst3 = jnp.concatenate([ei[1], pad_idx]).reshape(NW, NCH, CH)
    zeros2d = jnp.zeros((NGP, WSC), f32)
    mass_c = mass.reshape(NV, 1)
    evals_c = evals.reshape(KEV, 1)
    llW_pad = jnp.zeros((WID, 128), f32).at[:, :2].set(ll_W)
    llb_pad = jnp.zeros((1, 128), f32).at[:, :2].set(ll_b)
    b_row = lambda b: b.reshape(1, -1)

    degp = _sc_scatter(jnp.ones((NGP, WSC), f32), src3, dst3, zeros2d)

    rbf, rsum, csum = _rbf_pass(vertices.astype(f32), gp_pad)
    dx, gx, norm = _init_pass(sx_pad, fl1W_pad, b_row(fl1_b),
                              gx_in_pad, fl2_W, b_row(fl2_b), degp)

    for i in range(NB):
        dx = _diff_block(dx, evecs, mass_c, evals_c, b_row(T[i]),
                         DW0[i], b_row(Db0[i]), DW1[i], b_row(Db1[i]))
        y1 = _gin_pass(rbf, dx, gx, csum, norm, GW1[i])
        zp1 = _sc_scatter(y1, src3, dst3, zeros2d)
        y2 = _h_pass(zp1, y1, norm, b_row(Gb1[i]), GW2[i])
        zp2 = _sc_scatter(y2, src3, dst3, zeros2d)
        dx, gx = _dupd_pass(rbf, zp2, y2, norm, b_row(Gb2[i]), rsum, dx)

    out128 = _final_pass(dx, llW_pad, llb_pad)
    return out128[:, :2]
